# Initial kernel scaffold; baseline (speedup 1.0000x reference)
import jax
import jax.numpy as jnp
from jax.experimental import pallas as pl


def kernel(x, edge_index, W0l, b0, W0r, g0, be0, W1l, b1, W1r, g1, be1, W2l, b2, W2r):
    raise NotImplementedError("write your pallas kernel here")


# speedup vs baseline: 3.5326x; 3.5326x over previous
import functools

import jax
import jax.numpy as jnp
from jax import lax
from jax.experimental import pallas as pl
from jax.experimental.pallas import tpu as pltpu
from jax.experimental.pallas import tpu_sc as plsc

N = 10000
E = 320000
DIN = 128
DH = 256
DOUT = 128

DW = 128
K = 128
ROWS = 2560
EPAD = ROWS * K
NPAD = 10240
TSLICE = NPAD // 16
NB = 2


@functools.lru_cache(maxsize=None)
def _make_segsum(edgesplit: bool):
    mesh = plsc.VectorSubcoreMesh(core_axis_name="c", subcore_axis_name="s")
    rpt = ROWS // 32 if edgesplit else ROWS // 16

    blk = 16

    def body(tab, srcs, dsts, out, sidx, didx, rows, acc, gsem, ssem):
        c = lax.axis_index("c")
        s = lax.axis_index("s")
        row0 = (c * 16 + s) * rpt if edgesplit else s * rpt

        def zrow(i, carry):
            for j in range(DW // 16):
                rows[0, i, pl.ds(j * 16, 16)] = jnp.zeros((16,), jnp.float32)
            return carry
        lax.fori_loop(0, K, zrow, 0)
        for m in range(TSLICE // K):
            pltpu.sync_copy(rows.at[0], acc.at[pl.ds(s * TSLICE + m * K, K)])
        plsc.subcore_barrier()

        def block(t, carry):
            r0 = row0 + t * blk
            if edgesplit:
                pltpu.sync_copy(srcs.at[pl.ds(r0, blk)], sidx)
            else:
                pltpu.sync_copy(srcs.at[c, pl.ds(r0, blk)], sidx)
            pltpu.sync_copy(dsts.at[pl.ds(r0, blk)], didx)

            for b in range(NB):
                pltpu.async_copy(tab.at[sidx.at[b]], rows.at[b], gsem.at[b])

            ng = blk // NB

            def group(g, carry2):
                for b in range(NB):
                    j = g * NB + b
                    pltpu.make_async_copy(
                        tab.at[sidx.at[j]], rows.at[b], gsem.at[b]).wait()
                    pltpu.async_copy(
                        rows.at[b], acc.at[didx.at[j]], ssem.at[b], add=True)

                    @pl.when(g < ng - 1)
                    def _():
                        pltpu.make_async_copy(
                            rows.at[b], acc.at[didx.at[j]], ssem.at[b]).wait()
                        pltpu.async_copy(
                            tab.at[sidx.at[j + NB]], rows.at[b], gsem.at[b])
                return carry2
            lax.fori_loop(0, ng, group, 0)

            for b in range(NB):
                j = blk - NB + b
                pltpu.make_async_copy(
                    rows.at[b], acc.at[didx.at[j]], ssem.at[b]).wait()
            return carry
        lax.fori_loop(0, rpt // blk, block, 0)
        plsc.subcore_barrier()
        pltpu.sync_copy(acc.at[pl.ds(s * TSLICE, TSLICE)],
                        out.at[c, pl.ds(s * TSLICE, TSLICE)])

    return pl.kernel(
        body,
        out_type=jax.ShapeDtypeStruct((2, NPAD, DW), jnp.float32),
        mesh=mesh,
        scratch_types=[
            pltpu.VMEM((blk, K), jnp.int32),
            pltpu.VMEM((blk, K), jnp.int32),
            pltpu.VMEM((NB, K, DW), jnp.float32),
            pltpu.VMEM_SHARED((NPAD, DW), jnp.float32),
            pltpu.SemaphoreType.DMA((NB,)),
            pltpu.SemaphoreType.DMA((NB,)),
        ],
    )


def _make_degree():
    mesh = plsc.VectorSubcoreMesh(core_axis_name="c", subcore_axis_name="s")
    rpt = ROWS // 32

    def body(dsts, out, didx, ones, acc, sem):
        c = lax.axis_index("c")
        s = lax.axis_index("s")
        row0 = (c * 16 + s) * rpt
        pltpu.sync_copy(dsts.at[pl.ds(row0, rpt)], didx)

        def fill(i, carry):
            for j in range(DW // 16):
                ones[0, i, pl.ds(j * 16, 16)] = jnp.ones((16,), jnp.float32)
                ones[1, i, pl.ds(j * 16, 16)] = jnp.zeros((16,), jnp.float32)
            return carry
        lax.fori_loop(0, K, fill, 0)
        for m in range(TSLICE // K):
            pltpu.sync_copy(ones.at[1], acc.at[pl.ds(s * TSLICE + m * K, K)])
        plsc.subcore_barrier()

        for g in range(rpt // 8):
            for b in range(8):
                j = g * 8 + b
                pltpu.async_copy(ones.at[0], acc.at[didx.at[j]],
                                 sem.at[b], add=True)
            for b in range(8):
                j = g * 8 + b
                pltpu.make_async_copy(ones.at[0], acc.at[didx.at[j]],
                                      sem.at[b]).wait()
        plsc.subcore_barrier()
        pltpu.sync_copy(acc.at[pl.ds(s * TSLICE, TSLICE)],
                        out.at[c, pl.ds(s * TSLICE, TSLICE)])

    return pl.kernel(
        body,
        out_type=jax.ShapeDtypeStruct((2, NPAD, DW), jnp.float32),
        mesh=mesh,
        scratch_types=[
            pltpu.VMEM((rpt, K), jnp.int32),
            pltpu.VMEM((2, K, DW), jnp.float32),
            pltpu.VMEM_SHARED((NPAD, DW), jnp.float32),
            pltpu.SemaphoreType.DMA((8,)),
        ],
    )


_R = 2000


def _row_spec(d):
    return pl.BlockSpec((_R, d), lambda i: (i, 0))


def _full_spec(a, b):
    return pl.BlockSpec((a, b), lambda i: (0, 0))


def _l0_body(agga, aggb, deg, x, wl, b0, wr, g0, be0, o_ref):
    dinv = 1.0 / jnp.maximum(deg[...], 1.0)
    mean = (agga[...] + aggb[...]) * dinv
    y = (jnp.dot(mean, wl[...], preferred_element_type=jnp.float32)
         + jnp.dot(x[...], wr[...], preferred_element_type=jnp.float32)
         + b0[...])
    scale = g0[...] / jnp.sqrt(1.0 + 1e-5)
    o_ref[...] = jnp.maximum(y * scale + be0[...], 0.0)


def _l1_body(agg, deg, h, wl, b1, wr, g1, be1, w2l, h2_ref, p_ref):
    dinv = 1.0 / jnp.maximum(deg[...], 1.0)
    mean = agg[...] * dinv
    y = (jnp.dot(mean, wl[...], preferred_element_type=jnp.float32)
         + jnp.dot(h[...], wr[...], preferred_element_type=jnp.float32)
         + b1[...])
    scale = g1[...] / jnp.sqrt(1.0 + 1e-5)
    h2 = h[...] + jnp.maximum(y * scale + be1[...], 0.0)
    h2_ref[...] = h2
    p_ref[...] = jnp.dot(h2, w2l[...], preferred_element_type=jnp.float32)


def _l2_body(agga, aggb, deg, h2, wr, b2, o_ref):
    dinv = 1.0 / jnp.maximum(deg[...], 1.0)
    o_ref[...] = ((agga[...] + aggb[...]) * dinv + b2[...]
                  + jnp.dot(h2[...], wr[...], preferred_element_type=jnp.float32))


def _layer0(agga, aggb, deg, x, wl, b0, wr, g0, be0):
    return pl.pallas_call(
        _l0_body,
        grid=(N // _R,),
        in_specs=[_row_spec(DIN), _row_spec(DIN),
                  pl.BlockSpec((_R, 1), lambda i: (i, 0)),
                  _row_spec(DIN), _full_spec(DIN, DH), _full_spec(1, DH),
                  _full_spec(DIN, DH), _full_spec(1, DH), _full_spec(1, DH)],
        out_specs=_row_spec(DH),
        out_shape=jax.ShapeDtypeStruct((N, DH), jnp.float32),
    )(agga, aggb, deg, x, wl, b0, wr, g0, be0)


def _layer1(agg, deg, h, wl, b1, wr, g1, be1, w2l):
    return pl.pallas_call(
        _l1_body,
        grid=(N // _R,),
        in_specs=[_row_spec(DH), pl.BlockSpec((_R, 1), lambda i: (i, 0)),
                  _row_spec(DH), _full_spec(DH, DH), _full_spec(1, DH),
                  _full_spec(DH, DH), _full_spec(1, DH), _full_spec(1, DH),
                  _full_spec(DH, DOUT)],
        out_specs=[_row_spec(DH), _row_spec(DOUT)],
        out_shape=[jax.ShapeDtypeStruct((N, DH), jnp.float32),
                   jax.ShapeDtypeStruct((N, DOUT), jnp.float32)],
    )(agg, deg, h, wl, b1, wr, g1, be1, w2l)


def _layer2(agga, aggb, deg, h2, wr, b2):
    return pl.pallas_call(
        _l2_body,
        grid=(N // _R,),
        in_specs=[_row_spec(DOUT), _row_spec(DOUT),
                  pl.BlockSpec((_R, 1), lambda i: (i, 0)),
                  _row_spec(DH), _full_spec(DH, DOUT), _full_spec(1, DOUT)],
        out_specs=_row_spec(DOUT),
        out_shape=jax.ShapeDtypeStruct((N, DOUT), jnp.float32),
    )(agga, aggb, deg, h2, wr, b2)


def kernel(x, edge_index, W0l, b0, W0r, g0, be0, W1l, b1, W1r, g1, be1,
           W2l, b2, W2r):
    src = edge_index[0].astype(jnp.int32)
    dst = edge_index[1].astype(jnp.int32)
    npad = EPAD - E
    src_p = jnp.concatenate([src, jnp.zeros((npad,), jnp.int32)])
    dst_p = jnp.concatenate([dst, jnp.full((npad,), N, jnp.int32)])
    srcs = src_p.reshape(ROWS, K)
    dsts = dst_p.reshape(ROWS, K)
    srcs2 = jnp.stack([srcs, srcs + N])

    deg2 = _make_degree()(dsts)
    deg = deg2[0, :N, :1] + deg2[1, :N, :1]

    b0r = b0.reshape(1, DH); g0r = g0.reshape(1, DH); be0r = be0.reshape(1, DH)
    b1r = b1.reshape(1, DH); g1r = g1.reshape(1, DH); be1r = be1.reshape(1, DH)
    b2r = b2.reshape(1, DOUT)

    agg0 = _make_segsum(True)(x, srcs, dsts)
    h = _layer0(agg0[0, :N], agg0[1, :N], deg, x, W0l, b0r, W0r, g0r, be0r)

    tab1 = jnp.concatenate([h[:, :DW], h[:, DW:]], axis=0)
    agg1s = _make_segsum(False)(tab1, srcs2, dsts)
    agg1 = jnp.concatenate([agg1s[0, :N], agg1s[1, :N]], axis=1)
    h2, p = _layer1(agg1, deg, h, W1l, b1r, W1r, g1r, be1r, W2l)

    agg2 = _make_segsum(True)(p, srcs, dsts)
    out = _layer2(agg2[0, :N], agg2[1, :N], deg, h2, W2r, b2r)
    return out


# speedup vs baseline: 3.7671x; 1.0664x over previous
import functools

import jax
import jax.numpy as jnp
from jax import lax
from jax.experimental import pallas as pl
from jax.experimental.pallas import tpu as pltpu
from jax.experimental.pallas import tpu_sc as plsc

N = 10000
E = 320000
DIN = 128
DH = 256
DOUT = 128

DW = 128
K = 128
ROWS = 2560
EPAD = ROWS * K
NPAD = 10240
TSLICE = NPAD // 16
NB = 2


@functools.lru_cache(maxsize=None)
def _make_segsum(edgesplit: bool):
    mesh = plsc.VectorSubcoreMesh(core_axis_name="c", subcore_axis_name="s")
    rpt = ROWS // 32 if edgesplit else ROWS // 16

    blk = 16

    def body(tab, srcs, dsts, out, sidx, didx, rows, acc, gsem, ssem):
        c = lax.axis_index("c")
        s = lax.axis_index("s")
        row0 = (c * 16 + s) * rpt if edgesplit else s * rpt

        def zrow(i, carry):
            for j in range(DW // 16):
                rows[0, i, pl.ds(j * 16, 16)] = jnp.zeros((16,), jnp.float32)
            return carry
        lax.fori_loop(0, K, zrow, 0)
        for m in range(TSLICE // K):
            pltpu.sync_copy(rows.at[0], acc.at[pl.ds(s * TSLICE + m * K, K)])
        plsc.subcore_barrier()

        def block(t, carry):
            r0 = row0 + t * blk
            pltpu.sync_copy(srcs.at[c, pl.ds(r0, blk)], sidx)
            pltpu.sync_copy(dsts.at[pl.ds(r0, blk)], didx)

            for b in range(NB):
                pltpu.async_copy(tab.at[sidx.at[b]], rows.at[b], gsem.at[b])

            ng = blk // NB

            def group(g, carry2):
                for b in range(NB):
                    j = g * NB + b
                    pltpu.make_async_copy(
                        tab.at[sidx.at[j]], rows.at[b], gsem.at[b]).wait()
                    pltpu.async_copy(
                        rows.at[b], acc.at[didx.at[j]], ssem.at[b], add=True)

                    @pl.when(g < ng - 1)
                    def _():
                        pltpu.make_async_copy(
                            rows.at[b], acc.at[didx.at[j]], ssem.at[b]).wait()
                        pltpu.async_copy(
                            tab.at[sidx.at[j + NB]], rows.at[b], gsem.at[b])
                return carry2
            lax.fori_loop(0, ng, group, 0)

            for b in range(NB):
                j = blk - NB + b
                pltpu.make_async_copy(
                    rows.at[b], acc.at[didx.at[j]], ssem.at[b]).wait()
            return carry
        lax.fori_loop(0, rpt // blk, block, 0)
        plsc.subcore_barrier()
        pltpu.sync_copy(acc.at[pl.ds(s * TSLICE, TSLICE)],
                        out.at[c, pl.ds(s * TSLICE, TSLICE)])

    return pl.kernel(
        body,
        out_type=jax.ShapeDtypeStruct((2, NPAD, DW), jnp.float32),
        mesh=mesh,
        scratch_types=[
            pltpu.VMEM((blk, K), jnp.int32),
            pltpu.VMEM((blk, K), jnp.int32),
            pltpu.VMEM((NB, K, DW), jnp.float32),
            pltpu.VMEM_SHARED((NPAD, DW), jnp.float32),
            pltpu.SemaphoreType.DMA((NB,)),
            pltpu.SemaphoreType.DMA((NB,)),
        ],
    )


def _make_degree():
    mesh = plsc.VectorSubcoreMesh(core_axis_name="c", subcore_axis_name="s")
    rpt = ROWS // 32

    def body(dsts, out, didx, ones, acc, sem):
        c = lax.axis_index("c")
        s = lax.axis_index("s")
        row0 = (c * 16 + s) * rpt
        pltpu.sync_copy(dsts.at[pl.ds(row0, rpt)], didx)

        def fill(i, carry):
            for j in range(DW // 16):
                ones[0, i, pl.ds(j * 16, 16)] = jnp.ones((16,), jnp.float32)
                ones[1, i, pl.ds(j * 16, 16)] = jnp.zeros((16,), jnp.float32)
            return carry
        lax.fori_loop(0, K, fill, 0)
        for m in range(TSLICE // K):
            pltpu.sync_copy(ones.at[1], acc.at[pl.ds(s * TSLICE + m * K, K)])
        plsc.subcore_barrier()

        for g in range(rpt // 8):
            for b in range(8):
                j = g * 8 + b
                pltpu.async_copy(ones.at[0], acc.at[didx.at[j]],
                                 sem.at[b], add=True)
            for b in range(8):
                j = g * 8 + b
                pltpu.make_async_copy(ones.at[0], acc.at[didx.at[j]],
                                      sem.at[b]).wait()
        plsc.subcore_barrier()
        pltpu.sync_copy(acc.at[pl.ds(s * TSLICE, TSLICE)],
                        out.at[c, pl.ds(s * TSLICE, TSLICE)])

    return pl.kernel(
        body,
        out_type=jax.ShapeDtypeStruct((2, NPAD, DW), jnp.float32),
        mesh=mesh,
        scratch_types=[
            pltpu.VMEM((rpt, K), jnp.int32),
            pltpu.VMEM((2, K, DW), jnp.float32),
            pltpu.VMEM_SHARED((NPAD, DW), jnp.float32),
            pltpu.SemaphoreType.DMA((8,)),
        ],
    )


_R = 2000


def _row_spec(d):
    return pl.BlockSpec((_R, d), lambda i: (i, 0))


def _full_spec(a, b):
    return pl.BlockSpec((a, b), lambda i: (0, 0))


def _l0_body(agga, aggb, deg, x, wl, b0, wr, g0, be0, o_ref):
    dinv = 1.0 / jnp.maximum(deg[...], 1.0)
    mean = (agga[...] + aggb[...]) * dinv
    y = (jnp.dot(mean, wl[...], preferred_element_type=jnp.float32)
         + jnp.dot(x[...], wr[...], preferred_element_type=jnp.float32)
         + b0[...])
    scale = g0[...] / jnp.sqrt(1.0 + 1e-5)
    o_ref[...] = jnp.maximum(y * scale + be0[...], 0.0)


def _l1_body(agg, deg, h, wl, b1, wr, g1, be1, w2l, h2_ref, p_ref):
    dinv = 1.0 / jnp.maximum(deg[...], 1.0)
    mean = agg[...] * dinv
    y = (jnp.dot(mean, wl[...], preferred_element_type=jnp.float32)
         + jnp.dot(h[...], wr[...], preferred_element_type=jnp.float32)
         + b1[...])
    scale = g1[...] / jnp.sqrt(1.0 + 1e-5)
    h2 = h[...] + jnp.maximum(y * scale + be1[...], 0.0)
    h2_ref[...] = h2
    p_ref[...] = jnp.dot(h2, w2l[...], preferred_element_type=jnp.float32)


def _l2_body(agga, aggb, deg, h2, wr, b2, o_ref):
    dinv = 1.0 / jnp.maximum(deg[...], 1.0)
    o_ref[...] = ((agga[...] + aggb[...]) * dinv + b2[...]
                  + jnp.dot(h2[...], wr[...], preferred_element_type=jnp.float32))


def _layer0(agga, aggb, deg, x, wl, b0, wr, g0, be0):
    return pl.pallas_call(
        _l0_body,
        grid=(N // _R,),
        in_specs=[_row_spec(DIN), _row_spec(DIN),
                  pl.BlockSpec((_R, 1), lambda i: (i, 0)),
                  _row_spec(DIN), _full_spec(DIN, DH), _full_spec(1, DH),
                  _full_spec(DIN, DH), _full_spec(1, DH), _full_spec(1, DH)],
        out_specs=_row_spec(DH),
        out_shape=jax.ShapeDtypeStruct((N, DH), jnp.float32),
    )(agga, aggb, deg, x, wl, b0, wr, g0, be0)


def _layer1(agg, deg, h, wl, b1, wr, g1, be1, w2l):
    return pl.pallas_call(
        _l1_body,
        grid=(N // _R,),
        in_specs=[_row_spec(DH), pl.BlockSpec((_R, 1), lambda i: (i, 0)),
                  _row_spec(DH), _full_spec(DH, DH), _full_spec(1, DH),
                  _full_spec(DH, DH), _full_spec(1, DH), _full_spec(1, DH),
                  _full_spec(DH, DOUT)],
        out_specs=[_row_spec(DH), _row_spec(DOUT)],
        out_shape=[jax.ShapeDtypeStruct((N, DH), jnp.float32),
                   jax.ShapeDtypeStruct((N, DOUT), jnp.float32)],
    )(agg, deg, h, wl, b1, wr, g1, be1, w2l)


def _layer2(agga, aggb, deg, h2, wr, b2):
    return pl.pallas_call(
        _l2_body,
        grid=(N // _R,),
        in_specs=[_row_spec(DOUT), _row_spec(DOUT),
                  pl.BlockSpec((_R, 1), lambda i: (i, 0)),
                  _row_spec(DH), _full_spec(DH, DOUT), _full_spec(1, DOUT)],
        out_specs=_row_spec(DOUT),
        out_shape=jax.ShapeDtypeStruct((N, DOUT), jnp.float32),
    )(agga, aggb, deg, h2, wr, b2)


def kernel(x, edge_index, W0l, b0, W0r, g0, be0, W1l, b1, W1r, g1, be1,
           W2l, b2, W2r):
    src = edge_index[0].astype(jnp.int32)
    dst = edge_index[1].astype(jnp.int32)
    npad = EPAD - E
    src_p = jnp.concatenate([src, jnp.zeros((npad,), jnp.int32)])
    dst_p = jnp.concatenate([dst, jnp.full((npad,), N, jnp.int32)])
    srcs = src_p.reshape(ROWS, K)
    dsts = dst_p.reshape(ROWS, K)
    srcs2 = jnp.stack([srcs, srcs + N])

    deg2 = _make_degree()(dsts)
    deg = deg2[0, :N, :1] + deg2[1, :N, :1]

    b0r = b0.reshape(1, DH); g0r = g0.reshape(1, DH); be0r = be0.reshape(1, DH)
    b1r = b1.reshape(1, DH); g1r = g1.reshape(1, DH); be1r = be1.reshape(1, DH)
    b2r = b2.reshape(1, DOUT)

    tab0 = jnp.concatenate([x, x], axis=0)
    agg0 = _make_segsum(True)(tab0, srcs2, dsts)
    h = _layer0(agg0[0, :N], agg0[1, :N], deg, x, W0l, b0r, W0r, g0r, be0r)

    tab1 = jnp.concatenate([h[:, :DW], h[:, DW:]], axis=0)
    agg1s = _make_segsum(False)(tab1, srcs2, dsts)
    agg1 = jnp.concatenate([agg1s[0, :N], agg1s[1, :N]], axis=1)
    h2, p = _layer1(agg1, deg, h, W1l, b1r, W1r, g1r, be1r, W2l)

    tab2 = jnp.concatenate([p, p], axis=0)
    agg2 = _make_segsum(True)(tab2, srcs2, dsts)
    out = _layer2(agg2[0, :N], agg2[1, :N], deg, h2, W2r, b2r)
    return out
